# baseline (speedup 1.0000x reference)
import dataclasses
import functools

import jax
import jax.numpy as jnp
from jax import lax
from jax.experimental import pallas as pl
from jax.experimental.pallas import tpu as pltpu
from jax.experimental.pallas import tpu_sc as plsc

_CH = 40
_NW = 32


def _sc_compiler_params():
    cp = pltpu.CompilerParams(use_tc_tiling_on_sc=False)
    if "needs_layout_passes" in pltpu.CompilerParams.__dataclass_fields__:
        cp = dataclasses.replace(cp, needs_layout_passes=False)
    return cp


def _sc_gather(embeddings, indices, lengths):
    d = embeddings.shape[1]
    n_sent, seq = indices.shape
    spw = n_sent // _NW
    nch_max = seq // _CH
    idx3 = indices.reshape(_NW, spw, nch_max, _CH)
    len2 = lengths.reshape(_NW, spw)
    mesh = plsc.VectorSubcoreMesh(core_axis_name="core", subcore_axis_name="subcore")

    @functools.partial(
        pl.kernel,
        out_type=jax.ShapeDtypeStruct((n_sent * seq, d), embeddings.dtype),
        mesh=mesh,
        scratch_types=[
            pltpu.VMEM((spw, nch_max, _CH), jnp.int32),
            pltpu.VMEM((seq, d), jnp.float32),
            pltpu.VMEM((seq, d), jnp.float32),
            pltpu.VMEM((_CH, d), jnp.float32),
            pltpu.VMEM((32,), jnp.int32),
            pltpu.VMEM((spw,), jnp.int32),
            pltpu.SemaphoreType.DMA,
            pltpu.SemaphoreType.DMA,
            pltpu.SemaphoreType.DMA,
            pltpu.SemaphoreType.DMA,
        ],
        compiler_params=_sc_compiler_params(),
    )
    def k(x_hbm, i_hbm, l_hbm, o_hbm, idx_v, buf0, buf1, pad_v, zidx, lvm,
          g0, g1, w0, w1):
        wid = lax.axis_index("subcore") * 2 + lax.axis_index("core")
        pltpu.sync_copy(i_hbm.at[wid], idx_v)
        pltpu.sync_copy(l_hbm.at[wid], lvm)
        zidx[pl.ds(0, 16)] = jnp.zeros((16,), jnp.int32)
        zidx[pl.ds(16, 16)] = jnp.zeros((16,), jnp.int32)
        pltpu.sync_copy(x_hbm.at[zidx.at[pl.ds(0, _CH)]], pad_v)

        bufs = (buf0, buf1)
        gsems = (g0, g1)
        wsems = (w0, w1)

        lane = lax.iota(jnp.int32, 16)

        def nch_at(vec, j):
            val = jnp.max(jnp.where(lane == j, vec, 0))
            return (val + (_CH - 1)) // _CH

        def fire_gathers(sl, b, nch):
            @pl.loop(0, nch)
            def _(ch):
                pltpu.async_copy(
                    x_hbm.at[idx_v.at[sl, ch]],
                    bufs[b].at[pl.ds(ch * _CH, _CH)],
                    gsems[b],
                )

        def drain_gathers(nch, b):
            @pl.loop(0, nch)
            def _(ch):
                pltpu.make_async_copy(
                    x_hbm.at[idx_v.at[0, 0]],
                    bufs[b].at[pl.ds(0, _CH)],
                    gsems[b],
                ).wait()

        def fire_writes(sl, b, nch):
            base = (wid * spw + sl) * seq

            @pl.loop(0, nch_max)
            def _(ch):
                dst = o_hbm.at[pl.ds(base + ch * _CH, _CH)]

                @pl.when(ch < nch)
                def _():
                    pltpu.async_copy(
                        bufs[b].at[pl.ds(ch * _CH, _CH)], dst, wsems[b]
                    )

                @pl.when(ch >= nch)
                def _():
                    pltpu.async_copy(pad_v, dst, wsems[b])

        def drain_writes(b):
            @pl.loop(0, nch_max)
            def _(ch):
                pltpu.make_async_copy(
                    bufs[b].at[pl.ds(0, _CH)],
                    o_hbm.at[pl.ds(0, _CH)],
                    wsems[b],
                ).wait()

        @pl.loop(0, spw, step=16)
        def _(o):
            lvec = lvm[pl.ds(o, 16)]
            lprev = lvm[pl.ds(jnp.maximum(o - 16, 0), 16)]
            nchs = [nch_at(lvec, j) for j in range(16)]
            nch_prev_w = nch_at(lprev, 15)
            for j in range(16):
                s = o + j
                b = j % 2
                nch_prev = nchs[j - 1] if j > 0 else nch_prev_w

                @pl.when(s >= 2)
                def _():
                    drain_writes(b)

                fire_gathers(s, b, nchs[j])

                @pl.when(s >= 1)
                def _():
                    drain_gathers(nch_prev, 1 - b)
                    fire_writes(s - 1, 1 - b, nch_prev)

        ltail = lvm[pl.ds(spw - 16, 16)]
        nch_tail = nch_at(ltail, 15)
        drain_gathers(nch_tail, 1)
        fire_writes(spw - 1, 1, nch_tail)
        drain_writes(0)
        drain_writes(1)

    return k(embeddings, idx3, len2)


def _tc_mask(lengths, batch, seq):
    def mk(len_ref, out_ref):
        pos = lax.broadcasted_iota(jnp.int32, out_ref.shape, 1)
        out_ref[...] = (pos < len_ref[...]).astype(jnp.int8)

    rows = 128
    return pl.pallas_call(
        mk,
        grid=(batch // rows,),
        in_specs=[pl.BlockSpec((rows, 1), lambda i: (i, 0))],
        out_specs=pl.BlockSpec((rows, seq), lambda i: (i, 0)),
        out_shape=jax.ShapeDtypeStruct((batch, seq), jnp.int8),
    )(lengths.reshape(batch, 1))


def kernel(indices, lengths, embeddings):
    batch, seq = indices.shape
    d = embeddings.shape[1]
    emb_flat = _sc_gather(embeddings, indices, lengths)
    emb_words = emb_flat.reshape(batch, seq, d)
    att_words = _tc_mask(lengths, batch, seq).astype(jnp.bool_)
    return (emb_words, att_words)


# speedup vs baseline: 1.6707x; 1.6707x over previous
import dataclasses
import functools

import jax
import jax.numpy as jnp
from jax import lax
from jax.experimental import pallas as pl
from jax.experimental.pallas import tpu as pltpu
from jax.experimental.pallas import tpu_sc as plsc

_CH = 10
_NW = 32


def _sc_compiler_params():
    cp = pltpu.CompilerParams(use_tc_tiling_on_sc=False)
    if "needs_layout_passes" in pltpu.CompilerParams.__dataclass_fields__:
        cp = dataclasses.replace(cp, needs_layout_passes=False)
    return cp


def _sc_gather(embeddings, indices, lengths):
    d = embeddings.shape[1]
    n_sent, seq = indices.shape
    spw = n_sent // _NW
    nch_max = seq // _CH
    idx3 = indices.reshape(_NW, spw, nch_max, _CH)
    len2 = lengths.reshape(_NW, spw)
    mesh = plsc.VectorSubcoreMesh(core_axis_name="core", subcore_axis_name="subcore")

    @functools.partial(
        pl.kernel,
        out_type=jax.ShapeDtypeStruct((n_sent * seq, d), embeddings.dtype),
        mesh=mesh,
        scratch_types=[
            pltpu.VMEM((spw, nch_max, _CH), jnp.int32),
            pltpu.VMEM((seq, d), jnp.float32),
            pltpu.VMEM((seq, d), jnp.float32),
            pltpu.VMEM((_CH, d), jnp.float32),
            pltpu.VMEM((32,), jnp.int32),
            pltpu.VMEM((spw,), jnp.int32),
            pltpu.SemaphoreType.DMA,
            pltpu.SemaphoreType.DMA,
            pltpu.SemaphoreType.DMA,
            pltpu.SemaphoreType.DMA,
        ],
        compiler_params=_sc_compiler_params(),
    )
    def k(x_hbm, i_hbm, l_hbm, o_hbm, idx_v, buf0, buf1, pad_v, zidx, lvm,
          g0, g1, w0, w1):
        wid = lax.axis_index("subcore") * 2 + lax.axis_index("core")
        pltpu.sync_copy(i_hbm.at[wid], idx_v)
        pltpu.sync_copy(l_hbm.at[wid], lvm)
        zidx[pl.ds(0, 16)] = jnp.zeros((16,), jnp.int32)
        zidx[pl.ds(16, 16)] = jnp.zeros((16,), jnp.int32)
        pltpu.sync_copy(x_hbm.at[zidx.at[pl.ds(0, _CH)]], pad_v)

        bufs = (buf0, buf1)
        gsems = (g0, g1)
        wsems = (w0, w1)

        lane = lax.iota(jnp.int32, 16)

        def nch_at(vec, j):
            val = jnp.max(jnp.where(lane == j, vec, 0))
            return (val + (_CH - 1)) // _CH

        def fire_gathers(sl, b, nch):
            @pl.loop(0, nch)
            def _(ch):
                pltpu.async_copy(
                    x_hbm.at[idx_v.at[sl, ch]],
                    bufs[b].at[pl.ds(ch * _CH, _CH)],
                    gsems[b],
                )

        def drain_gathers(nch, b):
            @pl.loop(0, nch)
            def _(ch):
                pltpu.make_async_copy(
                    x_hbm.at[idx_v.at[0, 0]],
                    bufs[b].at[pl.ds(0, _CH)],
                    gsems[b],
                ).wait()

        def fire_writes(sl, b, nch):
            base = (wid * spw + sl) * seq

            @pl.loop(0, nch_max)
            def _(ch):
                dst = o_hbm.at[pl.ds(base + ch * _CH, _CH)]

                @pl.when(ch < nch)
                def _():
                    pltpu.async_copy(
                        bufs[b].at[pl.ds(ch * _CH, _CH)], dst, wsems[b]
                    )

                @pl.when(ch >= nch)
                def _():
                    pltpu.async_copy(pad_v, dst, wsems[b])

        def drain_writes(b):
            @pl.loop(0, nch_max)
            def _(ch):
                pltpu.make_async_copy(
                    bufs[b].at[pl.ds(0, _CH)],
                    o_hbm.at[pl.ds(0, _CH)],
                    wsems[b],
                ).wait()

        @pl.loop(0, spw, step=16)
        def _(o):
            lvec = lvm[pl.ds(o, 16)]
            lprev = lvm[pl.ds(jnp.maximum(o - 16, 0), 16)]
            nchs = [nch_at(lvec, j) for j in range(16)]
            nch_prev_w = nch_at(lprev, 15)
            for j in range(16):
                s = o + j
                b = j % 2
                nch_prev = nchs[j - 1] if j > 0 else nch_prev_w

                @pl.when(s >= 2)
                def _():
                    drain_writes(b)

                fire_gathers(s, b, nchs[j])

                @pl.when(s >= 1)
                def _():
                    drain_gathers(nch_prev, 1 - b)
                    fire_writes(s - 1, 1 - b, nch_prev)

        ltail = lvm[pl.ds(spw - 16, 16)]
        nch_tail = nch_at(ltail, 15)
        drain_gathers(nch_tail, 1)
        fire_writes(spw - 1, 1, nch_tail)
        drain_writes(0)
        drain_writes(1)

    return k(embeddings, idx3, len2)


def _tc_mask(lengths, batch, seq):
    def mk(len_ref, out_ref):
        pos = lax.broadcasted_iota(jnp.int32, out_ref.shape, 1)
        out_ref[...] = (pos < len_ref[...]).astype(jnp.int8)

    rows = 128
    return pl.pallas_call(
        mk,
        grid=(batch // rows,),
        in_specs=[pl.BlockSpec((rows, 1), lambda i: (i, 0))],
        out_specs=pl.BlockSpec((rows, seq), lambda i: (i, 0)),
        out_shape=jax.ShapeDtypeStruct((batch, seq), jnp.int8),
    )(lengths.reshape(batch, 1))


def kernel(indices, lengths, embeddings):
    batch, seq = indices.shape
    d = embeddings.shape[1]
    emb_flat = _sc_gather(embeddings, indices, lengths)
    emb_words = emb_flat.reshape(batch, seq, d)
    att_words = _tc_mask(lengths, batch, seq).astype(jnp.bool_)
    return (emb_words, att_words)


# speedup vs baseline: 1.7741x; 1.0619x over previous
import dataclasses
import functools

import jax
import jax.numpy as jnp
from jax import lax
from jax.experimental import pallas as pl
from jax.experimental.pallas import tpu as pltpu
from jax.experimental.pallas import tpu_sc as plsc

_CH = 8
_NW = 32


def _sc_compiler_params():
    cp = pltpu.CompilerParams(use_tc_tiling_on_sc=False)
    if "needs_layout_passes" in pltpu.CompilerParams.__dataclass_fields__:
        cp = dataclasses.replace(cp, needs_layout_passes=False)
    return cp


def _sc_gather(embeddings, indices, lengths):
    d = embeddings.shape[1]
    n_sent, seq = indices.shape
    spw = n_sent // _NW
    nch_max = seq // _CH
    idx3 = indices.reshape(_NW, spw, nch_max, _CH)
    len2 = lengths.reshape(_NW, spw)
    mesh = plsc.VectorSubcoreMesh(core_axis_name="core", subcore_axis_name="subcore")

    @functools.partial(
        pl.kernel,
        out_type=jax.ShapeDtypeStruct((n_sent * seq, d), embeddings.dtype),
        mesh=mesh,
        scratch_types=[
            pltpu.VMEM((spw, nch_max, _CH), jnp.int32),
            pltpu.VMEM((seq, d), jnp.float32),
            pltpu.VMEM((seq, d), jnp.float32),
            pltpu.VMEM((_CH, d), jnp.float32),
            pltpu.VMEM((32,), jnp.int32),
            pltpu.VMEM((spw,), jnp.int32),
            pltpu.SemaphoreType.DMA,
            pltpu.SemaphoreType.DMA,
            pltpu.SemaphoreType.DMA,
            pltpu.SemaphoreType.DMA,
        ],
        compiler_params=_sc_compiler_params(),
    )
    def k(x_hbm, i_hbm, l_hbm, o_hbm, idx_v, buf0, buf1, pad_v, zidx, lvm,
          g0, g1, w0, w1):
        wid = lax.axis_index("subcore") * 2 + lax.axis_index("core")
        pltpu.sync_copy(i_hbm.at[wid], idx_v)
        pltpu.sync_copy(l_hbm.at[wid], lvm)
        zidx[pl.ds(0, 16)] = jnp.zeros((16,), jnp.int32)
        zidx[pl.ds(16, 16)] = jnp.zeros((16,), jnp.int32)
        pltpu.sync_copy(x_hbm.at[zidx.at[pl.ds(0, _CH)]], pad_v)

        bufs = (buf0, buf1)
        gsems = (g0, g1)
        wsems = (w0, w1)

        lane = lax.iota(jnp.int32, 16)

        def nch_at(vec, j):
            val = jnp.max(jnp.where(lane == j, vec, 0))
            return (val + (_CH - 1)) // _CH

        def fire_gathers(sl, b, nch):
            @pl.loop(0, nch)
            def _(ch):
                pltpu.async_copy(
                    x_hbm.at[idx_v.at[sl, ch]],
                    bufs[b].at[pl.ds(ch * _CH, _CH)],
                    gsems[b],
                )

        def drain_gathers(nch, b):
            @pl.loop(0, nch)
            def _(ch):
                pltpu.make_async_copy(
                    x_hbm.at[idx_v.at[0, 0]],
                    bufs[b].at[pl.ds(0, _CH)],
                    gsems[b],
                ).wait()

        def fire_writes(sl, b, nch):
            base = (wid * spw + sl) * seq

            @pl.loop(0, nch_max)
            def _(ch):
                dst = o_hbm.at[pl.ds(base + ch * _CH, _CH)]

                @pl.when(ch < nch)
                def _():
                    pltpu.async_copy(
                        bufs[b].at[pl.ds(ch * _CH, _CH)], dst, wsems[b]
                    )

                @pl.when(ch >= nch)
                def _():
                    pltpu.async_copy(pad_v, dst, wsems[b])

        def drain_writes(b):
            @pl.loop(0, nch_max)
            def _(ch):
                pltpu.make_async_copy(
                    bufs[b].at[pl.ds(0, _CH)],
                    o_hbm.at[pl.ds(0, _CH)],
                    wsems[b],
                ).wait()

        @pl.loop(0, spw, step=16)
        def _(o):
            lvec = lvm[pl.ds(o, 16)]
            lprev = lvm[pl.ds(jnp.maximum(o - 16, 0), 16)]
            nchs = [nch_at(lvec, j) for j in range(16)]
            nch_prev_w = nch_at(lprev, 15)
            for j in range(16):
                s = o + j
                b = j % 2
                nch_prev = nchs[j - 1] if j > 0 else nch_prev_w

                @pl.when(s >= 2)
                def _():
                    drain_writes(b)

                fire_gathers(s, b, nchs[j])

                @pl.when(s >= 1)
                def _():
                    drain_gathers(nch_prev, 1 - b)
                    fire_writes(s - 1, 1 - b, nch_prev)

        ltail = lvm[pl.ds(spw - 16, 16)]
        nch_tail = nch_at(ltail, 15)
        drain_gathers(nch_tail, 1)
        fire_writes(spw - 1, 1, nch_tail)
        drain_writes(0)
        drain_writes(1)

    return k(embeddings, idx3, len2)


def _tc_mask(lengths, batch, seq):
    def mk(len_ref, out_ref):
        pos = lax.broadcasted_iota(jnp.int32, out_ref.shape, 1)
        out_ref[...] = (pos < len_ref[...]).astype(jnp.int8)

    rows = 128
    return pl.pallas_call(
        mk,
        grid=(batch // rows,),
        in_specs=[pl.BlockSpec((rows, 1), lambda i: (i, 0))],
        out_specs=pl.BlockSpec((rows, seq), lambda i: (i, 0)),
        out_shape=jax.ShapeDtypeStruct((batch, seq), jnp.int8),
    )(lengths.reshape(batch, 1))


def kernel(indices, lengths, embeddings):
    batch, seq = indices.shape
    d = embeddings.shape[1]
    emb_flat = _sc_gather(embeddings, indices, lengths)
    emb_words = emb_flat.reshape(batch, seq, d)
    att_words = _tc_mask(lengths, batch, seq).astype(jnp.bool_)
    return (emb_words, att_words)


# speedup vs baseline: 1.8963x; 1.0689x over previous
import dataclasses
import functools

import jax
import jax.numpy as jnp
from jax import lax
from jax.experimental import pallas as pl
from jax.experimental.pallas import tpu as pltpu
from jax.experimental.pallas import tpu_sc as plsc

_CH = 5
_NW = 32


def _sc_compiler_params():
    cp = pltpu.CompilerParams(use_tc_tiling_on_sc=False)
    if "needs_layout_passes" in pltpu.CompilerParams.__dataclass_fields__:
        cp = dataclasses.replace(cp, needs_layout_passes=False)
    return cp


def _sc_gather(embeddings, indices, lengths):
    d = embeddings.shape[1]
    n_sent, seq = indices.shape
    spw = n_sent // _NW
    nch_max = seq // _CH
    idx3 = indices.reshape(_NW, spw, nch_max, _CH)
    len2 = lengths.reshape(_NW, spw)
    mesh = plsc.VectorSubcoreMesh(core_axis_name="core", subcore_axis_name="subcore")

    @functools.partial(
        pl.kernel,
        out_type=jax.ShapeDtypeStruct((n_sent * seq, d), embeddings.dtype),
        mesh=mesh,
        scratch_types=[
            pltpu.VMEM((spw, nch_max, _CH), jnp.int32),
            pltpu.VMEM((seq, d), jnp.float32),
            pltpu.VMEM((seq, d), jnp.float32),
            pltpu.VMEM((_CH, d), jnp.float32),
            pltpu.VMEM((32,), jnp.int32),
            pltpu.VMEM((spw,), jnp.int32),
            pltpu.SemaphoreType.DMA,
            pltpu.SemaphoreType.DMA,
            pltpu.SemaphoreType.DMA,
            pltpu.SemaphoreType.DMA,
        ],
        compiler_params=_sc_compiler_params(),
    )
    def k(x_hbm, i_hbm, l_hbm, o_hbm, idx_v, buf0, buf1, pad_v, zidx, lvm,
          g0, g1, w0, w1):
        wid = lax.axis_index("subcore") * 2 + lax.axis_index("core")
        pltpu.sync_copy(i_hbm.at[wid], idx_v)
        pltpu.sync_copy(l_hbm.at[wid], lvm)
        zidx[pl.ds(0, 16)] = jnp.zeros((16,), jnp.int32)
        zidx[pl.ds(16, 16)] = jnp.zeros((16,), jnp.int32)
        pltpu.sync_copy(x_hbm.at[zidx.at[pl.ds(0, _CH)]], pad_v)

        bufs = (buf0, buf1)
        gsems = (g0, g1)
        wsems = (w0, w1)

        lane = lax.iota(jnp.int32, 16)

        def nch_at(vec, j):
            val = jnp.max(jnp.where(lane == j, vec, 0))
            return (val + (_CH - 1)) // _CH

        def fire_gathers(sl, b, nch):
            @pl.loop(0, nch)
            def _(ch):
                pltpu.async_copy(
                    x_hbm.at[idx_v.at[sl, ch]],
                    bufs[b].at[pl.ds(ch * _CH, _CH)],
                    gsems[b],
                )

        def drain_gathers(nch, b):
            @pl.loop(0, nch)
            def _(ch):
                pltpu.make_async_copy(
                    x_hbm.at[idx_v.at[0, 0]],
                    bufs[b].at[pl.ds(0, _CH)],
                    gsems[b],
                ).wait()

        def fire_writes(sl, b, nch):
            base = (wid * spw + sl) * seq

            @pl.loop(0, nch_max)
            def _(ch):
                dst = o_hbm.at[pl.ds(base + ch * _CH, _CH)]

                @pl.when(ch < nch)
                def _():
                    pltpu.async_copy(
                        bufs[b].at[pl.ds(ch * _CH, _CH)], dst, wsems[b]
                    )

                @pl.when(ch >= nch)
                def _():
                    pltpu.async_copy(pad_v, dst, wsems[b])

        def drain_writes(b):
            @pl.loop(0, nch_max)
            def _(ch):
                pltpu.make_async_copy(
                    bufs[b].at[pl.ds(0, _CH)],
                    o_hbm.at[pl.ds(0, _CH)],
                    wsems[b],
                ).wait()

        @pl.loop(0, spw, step=16)
        def _(o):
            lvec = lvm[pl.ds(o, 16)]
            lprev = lvm[pl.ds(jnp.maximum(o - 16, 0), 16)]
            nchs = [nch_at(lvec, j) for j in range(16)]
            nch_prev_w = nch_at(lprev, 15)
            for j in range(16):
                s = o + j
                b = j % 2
                nch_prev = nchs[j - 1] if j > 0 else nch_prev_w

                @pl.when(s >= 2)
                def _():
                    drain_writes(b)

                fire_gathers(s, b, nchs[j])

                @pl.when(s >= 1)
                def _():
                    drain_gathers(nch_prev, 1 - b)
                    fire_writes(s - 1, 1 - b, nch_prev)

        ltail = lvm[pl.ds(spw - 16, 16)]
        nch_tail = nch_at(ltail, 15)
        drain_gathers(nch_tail, 1)
        fire_writes(spw - 1, 1, nch_tail)
        drain_writes(0)
        drain_writes(1)

    return k(embeddings, idx3, len2)


def _tc_mask(lengths, batch, seq):
    def mk(len_ref, out_ref):
        pos = lax.broadcasted_iota(jnp.int32, out_ref.shape, 1)
        out_ref[...] = (pos < len_ref[...]).astype(jnp.int8)

    rows = 128
    return pl.pallas_call(
        mk,
        grid=(batch // rows,),
        in_specs=[pl.BlockSpec((rows, 1), lambda i: (i, 0))],
        out_specs=pl.BlockSpec((rows, seq), lambda i: (i, 0)),
        out_shape=jax.ShapeDtypeStruct((batch, seq), jnp.int8),
    )(lengths.reshape(batch, 1))


def kernel(indices, lengths, embeddings):
    batch, seq = indices.shape
    d = embeddings.shape[1]
    emb_flat = _sc_gather(embeddings, indices, lengths)
    emb_words = emb_flat.reshape(batch, seq, d)
    att_words = _tc_mask(lengths, batch, seq).astype(jnp.bool_)
    return (emb_words, att_words)


# speedup vs baseline: 1.9043x; 1.0043x over previous
import dataclasses
import functools

import jax
import jax.numpy as jnp
from jax import lax
from jax.experimental import pallas as pl
from jax.experimental.pallas import tpu as pltpu
from jax.experimental.pallas import tpu_sc as plsc

_CH = 4
_NW = 32


def _sc_compiler_params():
    cp = pltpu.CompilerParams(use_tc_tiling_on_sc=False)
    if "needs_layout_passes" in pltpu.CompilerParams.__dataclass_fields__:
        cp = dataclasses.replace(cp, needs_layout_passes=False)
    return cp


def _sc_gather(embeddings, indices, lengths):
    d = embeddings.shape[1]
    n_sent, seq = indices.shape
    spw = n_sent // _NW
    nch_max = seq // _CH
    idx3 = indices.reshape(_NW, spw, nch_max, _CH)
    len2 = lengths.reshape(_NW, spw)
    mesh = plsc.VectorSubcoreMesh(core_axis_name="core", subcore_axis_name="subcore")

    @functools.partial(
        pl.kernel,
        out_type=jax.ShapeDtypeStruct((n_sent * seq, d), embeddings.dtype),
        mesh=mesh,
        scratch_types=[
            pltpu.VMEM((spw, nch_max, _CH), jnp.int32),
            pltpu.VMEM((seq, d), jnp.float32),
            pltpu.VMEM((seq, d), jnp.float32),
            pltpu.VMEM((_CH, d), jnp.float32),
            pltpu.VMEM((32,), jnp.int32),
            pltpu.VMEM((spw,), jnp.int32),
            pltpu.SemaphoreType.DMA,
            pltpu.SemaphoreType.DMA,
            pltpu.SemaphoreType.DMA,
            pltpu.SemaphoreType.DMA,
        ],
        compiler_params=_sc_compiler_params(),
    )
    def k(x_hbm, i_hbm, l_hbm, o_hbm, idx_v, buf0, buf1, pad_v, zidx, lvm,
          g0, g1, w0, w1):
        wid = lax.axis_index("subcore") * 2 + lax.axis_index("core")
        pltpu.sync_copy(i_hbm.at[wid], idx_v)
        pltpu.sync_copy(l_hbm.at[wid], lvm)
        zidx[pl.ds(0, 16)] = jnp.zeros((16,), jnp.int32)
        zidx[pl.ds(16, 16)] = jnp.zeros((16,), jnp.int32)
        pltpu.sync_copy(x_hbm.at[zidx.at[pl.ds(0, _CH)]], pad_v)

        bufs = (buf0, buf1)
        gsems = (g0, g1)
        wsems = (w0, w1)

        lane = lax.iota(jnp.int32, 16)

        def nch_at(vec, j):
            val = jnp.max(jnp.where(lane == j, vec, 0))
            return (val + (_CH - 1)) // _CH

        def fire_gathers(sl, b, nch):
            @pl.loop(0, nch)
            def _(ch):
                pltpu.async_copy(
                    x_hbm.at[idx_v.at[sl, ch]],
                    bufs[b].at[pl.ds(ch * _CH, _CH)],
                    gsems[b],
                )

        def drain_gathers(nch, b):
            @pl.loop(0, nch)
            def _(ch):
                pltpu.make_async_copy(
                    x_hbm.at[idx_v.at[0, 0]],
                    bufs[b].at[pl.ds(0, _CH)],
                    gsems[b],
                ).wait()

        def fire_writes(sl, b, nch):
            base = (wid * spw + sl) * seq

            @pl.loop(0, nch_max)
            def _(ch):
                dst = o_hbm.at[pl.ds(base + ch * _CH, _CH)]

                @pl.when(ch < nch)
                def _():
                    pltpu.async_copy(
                        bufs[b].at[pl.ds(ch * _CH, _CH)], dst, wsems[b]
                    )

                @pl.when(ch >= nch)
                def _():
                    pltpu.async_copy(pad_v, dst, wsems[b])

        def drain_writes(b):
            @pl.loop(0, nch_max)
            def _(ch):
                pltpu.make_async_copy(
                    bufs[b].at[pl.ds(0, _CH)],
                    o_hbm.at[pl.ds(0, _CH)],
                    wsems[b],
                ).wait()

        @pl.loop(0, spw, step=16)
        def _(o):
            lvec = lvm[pl.ds(o, 16)]
            lprev = lvm[pl.ds(jnp.maximum(o - 16, 0), 16)]
            nchs = [nch_at(lvec, j) for j in range(16)]
            nch_prev_w = nch_at(lprev, 15)
            for j in range(16):
                s = o + j
                b = j % 2
                nch_prev = nchs[j - 1] if j > 0 else nch_prev_w

                @pl.when(s >= 2)
                def _():
                    drain_writes(b)

                fire_gathers(s, b, nchs[j])

                @pl.when(s >= 1)
                def _():
                    drain_gathers(nch_prev, 1 - b)
                    fire_writes(s - 1, 1 - b, nch_prev)

        ltail = lvm[pl.ds(spw - 16, 16)]
        nch_tail = nch_at(ltail, 15)
        drain_gathers(nch_tail, 1)
        fire_writes(spw - 1, 1, nch_tail)
        drain_writes(0)
        drain_writes(1)

    return k(embeddings, idx3, len2)


def _tc_mask(lengths, batch, seq):
    def mk(len_ref, out_ref):
        pos = lax.broadcasted_iota(jnp.int32, out_ref.shape, 1)
        out_ref[...] = (pos < len_ref[...]).astype(jnp.int8)

    rows = 128
    return pl.pallas_call(
        mk,
        grid=(batch // rows,),
        in_specs=[pl.BlockSpec((rows, 1), lambda i: (i, 0))],
        out_specs=pl.BlockSpec((rows, seq), lambda i: (i, 0)),
        out_shape=jax.ShapeDtypeStruct((batch, seq), jnp.int8),
    )(lengths.reshape(batch, 1))


def kernel(indices, lengths, embeddings):
    batch, seq = indices.shape
    d = embeddings.shape[1]
    emb_flat = _sc_gather(embeddings, indices, lengths)
    emb_words = emb_flat.reshape(batch, seq, d)
    att_words = _tc_mask(lengths, batch, seq).astype(jnp.bool_)
    return (emb_words, att_words)


# speedup vs baseline: 1.9210x; 1.0087x over previous
import dataclasses
import functools

import jax
import jax.numpy as jnp
from jax import lax
from jax.experimental import pallas as pl
from jax.experimental.pallas import tpu as pltpu
from jax.experimental.pallas import tpu_sc as plsc

_CH = 2
_NW = 32


def _sc_compiler_params():
    cp = pltpu.CompilerParams(use_tc_tiling_on_sc=False)
    if "needs_layout_passes" in pltpu.CompilerParams.__dataclass_fields__:
        cp = dataclasses.replace(cp, needs_layout_passes=False)
    return cp


def _sc_gather(embeddings, indices, lengths):
    d = embeddings.shape[1]
    n_sent, seq = indices.shape
    spw = n_sent // _NW
    nch_max = seq // _CH
    idx3 = indices.reshape(_NW, spw, nch_max, _CH)
    len2 = lengths.reshape(_NW, spw)
    mesh = plsc.VectorSubcoreMesh(core_axis_name="core", subcore_axis_name="subcore")

    @functools.partial(
        pl.kernel,
        out_type=jax.ShapeDtypeStruct((n_sent * seq, d), embeddings.dtype),
        mesh=mesh,
        scratch_types=[
            pltpu.VMEM((spw, nch_max, _CH), jnp.int32),
            pltpu.VMEM((seq, d), jnp.float32),
            pltpu.VMEM((seq, d), jnp.float32),
            pltpu.VMEM((_CH, d), jnp.float32),
            pltpu.VMEM((32,), jnp.int32),
            pltpu.VMEM((spw,), jnp.int32),
            pltpu.SemaphoreType.DMA,
            pltpu.SemaphoreType.DMA,
            pltpu.SemaphoreType.DMA,
            pltpu.SemaphoreType.DMA,
        ],
        compiler_params=_sc_compiler_params(),
    )
    def k(x_hbm, i_hbm, l_hbm, o_hbm, idx_v, buf0, buf1, pad_v, zidx, lvm,
          g0, g1, w0, w1):
        wid = lax.axis_index("subcore") * 2 + lax.axis_index("core")
        pltpu.sync_copy(i_hbm.at[wid], idx_v)
        pltpu.sync_copy(l_hbm.at[wid], lvm)
        zidx[pl.ds(0, 16)] = jnp.zeros((16,), jnp.int32)
        zidx[pl.ds(16, 16)] = jnp.zeros((16,), jnp.int32)
        pltpu.sync_copy(x_hbm.at[zidx.at[pl.ds(0, _CH)]], pad_v)

        bufs = (buf0, buf1)
        gsems = (g0, g1)
        wsems = (w0, w1)

        lane = lax.iota(jnp.int32, 16)

        def nch_at(vec, j):
            val = jnp.max(jnp.where(lane == j, vec, 0))
            return (val + (_CH - 1)) // _CH

        def fire_gathers(sl, b, nch):
            @pl.loop(0, nch)
            def _(ch):
                pltpu.async_copy(
                    x_hbm.at[idx_v.at[sl, ch]],
                    bufs[b].at[pl.ds(ch * _CH, _CH)],
                    gsems[b],
                )

        def drain_gathers(nch, b):
            @pl.loop(0, nch)
            def _(ch):
                pltpu.make_async_copy(
                    x_hbm.at[idx_v.at[0, 0]],
                    bufs[b].at[pl.ds(0, _CH)],
                    gsems[b],
                ).wait()

        def fire_writes(sl, b, nch):
            base = (wid * spw + sl) * seq

            @pl.loop(0, nch_max)
            def _(ch):
                dst = o_hbm.at[pl.ds(base + ch * _CH, _CH)]

                @pl.when(ch < nch)
                def _():
                    pltpu.async_copy(
                        bufs[b].at[pl.ds(ch * _CH, _CH)], dst, wsems[b]
                    )

                @pl.when(ch >= nch)
                def _():
                    pltpu.async_copy(pad_v, dst, wsems[b])

        def drain_writes(b):
            @pl.loop(0, nch_max)
            def _(ch):
                pltpu.make_async_copy(
                    bufs[b].at[pl.ds(0, _CH)],
                    o_hbm.at[pl.ds(0, _CH)],
                    wsems[b],
                ).wait()

        @pl.loop(0, spw, step=16)
        def _(o):
            lvec = lvm[pl.ds(o, 16)]
            lprev = lvm[pl.ds(jnp.maximum(o - 16, 0), 16)]
            nchs = [nch_at(lvec, j) for j in range(16)]
            nch_prev_w = nch_at(lprev, 15)
            for j in range(16):
                s = o + j
                b = j % 2
                nch_prev = nchs[j - 1] if j > 0 else nch_prev_w

                @pl.when(s >= 2)
                def _():
                    drain_writes(b)

                fire_gathers(s, b, nchs[j])

                @pl.when(s >= 1)
                def _():
                    drain_gathers(nch_prev, 1 - b)
                    fire_writes(s - 1, 1 - b, nch_prev)

        ltail = lvm[pl.ds(spw - 16, 16)]
        nch_tail = nch_at(ltail, 15)
        drain_gathers(nch_tail, 1)
        fire_writes(spw - 1, 1, nch_tail)
        drain_writes(0)
        drain_writes(1)

    return k(embeddings, idx3, len2)


def _tc_mask(lengths, batch, seq):
    def mk(len_ref, out_ref):
        pos = lax.broadcasted_iota(jnp.int32, out_ref.shape, 1)
        out_ref[...] = (pos < len_ref[...]).astype(jnp.int8)

    rows = 128
    return pl.pallas_call(
        mk,
        grid=(batch // rows,),
        in_specs=[pl.BlockSpec((rows, 1), lambda i: (i, 0))],
        out_specs=pl.BlockSpec((rows, seq), lambda i: (i, 0)),
        out_shape=jax.ShapeDtypeStruct((batch, seq), jnp.int8),
    )(lengths.reshape(batch, 1))


def kernel(indices, lengths, embeddings):
    batch, seq = indices.shape
    d = embeddings.shape[1]
    emb_flat = _sc_gather(embeddings, indices, lengths)
    emb_words = emb_flat.reshape(batch, seq, d)
    att_words = _tc_mask(lengths, batch, seq).astype(jnp.bool_)
    return (emb_words, att_words)
